# baseline (speedup 1.0000x reference)
import functools

import jax
import jax.numpy as jnp
from jax import lax
from jax.experimental import pallas as pl
from jax.experimental.pallas import tpu as pltpu
from jax.experimental.pallas import tpu_sc as plsc

H, W = 2048, 2048
P = 64
N = 4096
PW = P + 8


def _patch_kernel(images, positions):
    info = plsc.get_sparse_core_info()
    nw = info.num_cores * info.num_subcores
    per_w = N // nw
    nbuf = 6

    mesh = plsc.VectorSubcoreMesh(core_axis_name="c", subcore_axis_name="s")

    @functools.partial(
        pl.kernel,
        mesh=mesh,
        compiler_params=pltpu.CompilerParams(
            use_tc_tiling_on_sc=False, needs_layout_passes=False
        ),
        out_type=jax.ShapeDtypeStruct((N, P, P), jnp.float32),
        scratch_types=[
            pltpu.VMEM((2 * per_w,), jnp.int32),
            pltpu.SMEM((per_w, 2), jnp.int32),
            pltpu.VMEM((nbuf, P, PW), jnp.float32),
            pltpu.VMEM((nbuf, P, P), jnp.float32),
            pltpu.SemaphoreType.DMA((nbuf,)),
            pltpu.SemaphoreType.DMA((nbuf,)),
        ],
    )
    def k(img_hbm, pos_hbm, out_hbm, pos_v, pos_s, buf, sbuf,
          in_sem, out_sem):
        wid = lax.axis_index("s") * info.num_cores + lax.axis_index("c")
        base = wid * per_w
        pltpu.sync_copy(pos_hbm.at[pl.ds(2 * base, 2 * per_w)], pos_v)
        for j in range(per_w // 8):
            v = pos_v[pl.ds(16 * j, 16)]
            for t in range(8):
                pos_s[8 * j + t, 0] = v[2 * t]
                pos_s[8 * j + t, 1] = v[2 * t + 1]

        lane = lax.iota(jnp.int32, 16)

        def start_in(i, slot):
            r = pos_s[i, 0]
            c = pos_s[i, 1]
            c8 = pl.multiple_of((c // 8) * 8, 8)
            pltpu.make_async_copy(
                img_hbm.at[pl.ds(r, P), pl.ds(c8, PW)],
                buf.at[slot],
                in_sem.at[slot],
            ).start()

        def wait_in(slot):
            pltpu.make_async_copy(
                img_hbm.at[pl.ds(0, P), pl.ds(0, PW)],
                buf.at[slot],
                in_sem.at[slot],
            ).wait()

        def start_out(i, slot):
            pltpu.make_async_copy(
                sbuf.at[slot], out_hbm.at[base + i], out_sem.at[slot]
            ).start()

        def wait_out(i, slot):
            pltpu.make_async_copy(
                sbuf.at[slot], out_hbm.at[base + i], out_sem.at[slot]
            ).wait()

        def shuffle(i, slot):
            s = pos_s[i, 1] % 8
            src = buf.at[slot]
            dst = sbuf.at[slot]
            cidx = [s + kk * 16 + lane for kk in range(P // 16)]

            oidx = [kk * 16 + lane for kk in range(P // 16)]

            def row_body(row, _):
                ridx = jnp.full((16,), row, dtype=jnp.int32)
                for kk in range(P // 16):
                    v = plsc.load_gather(src, [ridx, cidx[kk]])
                    plsc.store_scatter(dst, [ridx, oidx[kk]], v)
                return 0

            lax.fori_loop(0, P, row_body, 0)

        for i in range(nbuf):
            start_in(i, i)

        def body(i, _):
            slot = lax.rem(i, nbuf)
            wait_in(slot)

            @pl.when(i >= nbuf)
            def _():
                wait_out(i - nbuf, slot)

            shuffle(i, slot)
            start_out(i, slot)

            @pl.when(i + nbuf < per_w)
            def _():
                start_in(i + nbuf, slot)

            return 0

        lax.fori_loop(0, per_w, body, 0)
        for t in range(nbuf):
            i = per_w - nbuf + t
            wait_out(i, lax.rem(i, nbuf))

    return k(images, positions.reshape(-1))


def kernel(images, positions, widths):
    del widths
    return _patch_kernel(images, positions)


# speedup vs baseline: 1.4137x; 1.4137x over previous
import functools

import jax
import jax.numpy as jnp
from jax import lax
from jax.experimental import pallas as pl
from jax.experimental.pallas import tpu as pltpu
from jax.experimental.pallas import tpu_sc as plsc

H, W = 2048, 2048
P = 64
N = 4096
PW = P + 8
CH = 16
RH = 32
NT = N // 128


def _patch_kernel(images, positions):
    info = plsc.get_sparse_core_info()
    nw = info.num_cores * info.num_subcores
    per_w = N // nw
    nchunk = per_w // CH
    nhalf = 2 * nchunk

    mesh = plsc.VectorSubcoreMesh(core_axis_name="c", subcore_axis_name="s")

    @functools.partial(
        pl.kernel,
        mesh=mesh,
        compiler_params=pltpu.CompilerParams(
            use_tc_tiling_on_sc=False, needs_layout_passes=False
        ),
        out_type=jax.ShapeDtypeStruct((P, 8, NT, 8, 128), jnp.float32),
        scratch_types=[
            pltpu.VMEM((2 * per_w,), jnp.int32),
            pltpu.SMEM((per_w, 2), jnp.int32),
            pltpu.VMEM((2, CH, RH, PW), jnp.float32),
            pltpu.VMEM((RH, 8, 1, 8, CH), jnp.float32),
            pltpu.SemaphoreType.DMA((2,)),
            pltpu.SemaphoreType.DMA,
        ],
    )
    def k(img_hbm, pos_hbm, out_hbm, pos_v, pos_s, ibuf, tbuf, in_sem,
          out_sem):
        wid = lax.axis_index("s") * info.num_cores + lax.axis_index("c")
        base = wid * per_w
        pltpu.sync_copy(pos_hbm.at[pl.ds(2 * base, 2 * per_w)], pos_v)
        for j in range(per_w // 8):
            v = pos_v[pl.ds(16 * j, 16)]
            for t in range(8):
                pos_s[8 * j + t, 0] = v[2 * t]
                pos_s[8 * j + t, 1] = v[2 * t + 1]

        lane = lax.iota(jnp.int32, 16)
        jt_idx = [(kk * 16 + lane) // 8 for kk in range(P // 16)]
        jr_idx = lane % 8
        zero_idx = jnp.zeros((16,), dtype=jnp.int32)

        def start_in(h, slot):
            cbase = (h // 2) * CH
            roff = (h % 2) * RH
            for t in range(CH):
                r = pos_s[cbase + t, 0]
                c = pos_s[cbase + t, 1]
                c8 = pl.multiple_of((c // 8) * 8, 8)
                pltpu.make_async_copy(
                    img_hbm.at[pl.ds(r + roff, RH), pl.ds(c8, PW)],
                    ibuf.at[slot, t],
                    in_sem.at[slot],
                ).start()

        def wait_in(slot):
            for t in range(CH):
                pltpu.make_async_copy(
                    img_hbm.at[pl.ds(0, RH), pl.ds(0, PW)],
                    ibuf.at[slot, t],
                    in_sem.at[slot],
                ).wait()

        def out_dst(h):
            cbase = (h // 2) * CH
            roff = (h % 2) * RH
            return out_hbm.at[
                pl.ds(roff, RH),
                slice(None),
                pl.ds(wid, 1),
                slice(None),
                pl.ds(cbase, CH),
            ]

        def start_out(h):
            pltpu.make_async_copy(tbuf, out_dst(h), out_sem).start()

        def wait_out(h):
            pltpu.make_async_copy(tbuf, out_dst(h), out_sem).wait()

        def shuffle(h, slot):
            cbase = (h // 2) * CH

            def patch_body(t, _):
                s = pos_s[cbase + t, 1] % 8
                src = ibuf.at[slot, t]
                pn = jnp.full((16,), t, dtype=jnp.int32)
                cidx = [s + kk * 16 + lane for kk in range(P // 16)]

                def row_body(row, _):
                    ridx = jnp.full((16,), row, dtype=jnp.int32)
                    for kk in range(P // 16):
                        v = plsc.load_gather(src, [ridx, cidx[kk]])
                        plsc.store_scatter(
                            tbuf,
                            [ridx, jt_idx[kk], zero_idx, jr_idx, pn],
                            v,
                        )
                    return 0

                lax.fori_loop(0, RH, row_body, 0)
                return 0

            lax.fori_loop(0, CH, patch_body, 0)

        start_in(0, 0)

        def body(h, _):
            slot = lax.rem(h, 2)
            nslot = lax.rem(h + 1, 2)

            @pl.when(h + 1 < nhalf)
            def _():
                start_in(h + 1, nslot)

            wait_in(slot)

            @pl.when(h >= 1)
            def _():
                wait_out(h - 1)

            shuffle(h, slot)
            start_out(h)
            return 0

        lax.fori_loop(0, nhalf, body, 0)
        wait_out(nhalf - 1)

    return k(images, positions.reshape(-1))


def kernel(images, positions, widths):
    del widths
    out5 = _patch_kernel(images, positions)
    return out5.transpose(2, 4, 0, 1, 3).reshape(N, P, P)
